# baseline (speedup 1.0000x reference)
import functools

import jax
import jax.numpy as jnp
from jax import lax
from jax.experimental import pallas as pl
from jax.experimental.pallas import tpu as pltpu
from jax.experimental.pallas import tpu_sc as plsc

L = 16
NC = 2
NS = 16
NW = NC * NS


def _iota16():
    return lax.iota(jnp.int32, L)


def _make_relayout(V, D):
    n_vt = (V + 127) // 128
    rows_out = n_vt * D
    n_pair = n_vt // 2
    t_rounds = (n_pair // NW + 3) // 2 * 2
    mesh = plsc.VectorSubcoreMesh(core_axis_name="c", subcore_axis_name="s")

    @functools.partial(
        pl.kernel,
        mesh=mesh,
        out_type=jax.ShapeDtypeStruct((rows_out, 128), jnp.float32),
        scratch_types=[
            pltpu.VMEM((D, 256), jnp.float32),
            pltpu.VMEM((D, 256), jnp.float32),
            pltpu.VMEM((2 * D, 128), jnp.float32),
            pltpu.VMEM((2 * D, 128), jnp.float32),
            pltpu.SemaphoreType.DMA,
            pltpu.SemaphoreType.DMA,
            pltpu.SemaphoreType.DMA,
            pltpu.SemaphoreType.DMA,
        ],
        compiler_params=pltpu.CompilerParams(
            use_tc_tiling_on_sc=True, needs_layout_passes=False
        ),
    )
    def relayout(tt_hbm, out_hbm, src0, src1, dst0, dst1,
                 isem0, isem1, osem0, osem1):
        wid = lax.axis_index("s") * NC + lax.axis_index("c")
        srcs, dsts = (src0, src1), (dst0, dst1)
        isems, osems = (isem0, isem1), (osem0, osem1)

        iot = _iota16()
        halfrow = [jnp.full((L,), 8 * b, jnp.int32) + (iot // 2)
                   for b in range(8)]
        lconst = [iot + (16 * b) for b in range(16)]
        parityd = (iot & 1) * D

        def fire_in(q, p):
            @pl.when(q < n_pair)
            def _():
                pltpu.async_copy(
                    tt_hbm.at[:, pl.ds(q * 256, 256)], srcs[p], isems[p]
                )

            @pl.when(q == n_pair)
            def _():
                pltpu.async_copy(
                    tt_hbm.at[:, pl.ds(q * 256, 128)],
                    srcs[p].at[:, pl.ds(0, 128)], isems[p]
                )

        def wait_in(q, p):
            @pl.when(q < n_pair)
            def _():
                pltpu.make_async_copy(
                    tt_hbm.at[:, pl.ds(q * 256, 256)], srcs[p], isems[p]
                ).wait()

            @pl.when(q == n_pair)
            def _():
                pltpu.make_async_copy(
                    tt_hbm.at[:, pl.ds(q * 256, 128)],
                    srcs[p].at[:, pl.ds(0, 128)], isems[p]
                ).wait()

        def full_out(q, p):
            return pltpu.make_async_copy(
                dsts[p], out_hbm.at[pl.ds(q * 2 * D, 2 * D)], osems[p]
            )

        fire_in(wid, 0)

        def round_body(u, carry):
            for p in range(2):
                t = 2 * u + p
                q = wid + t * NW

                fire_in(q + NW, 1 - p)

                @pl.when(q <= n_pair)
                def _(t=t, q=q, p=p):
                    wait_in(q, p)

                    @pl.when(t >= 2)
                    def _():
                        full_out(q - 2 * NW, p).wait()

                    src, dst = srcs[p], dsts[p]

                    def m_body(m, c2):
                        rotm = (iot + m) & 15
                        for a in range(D // L):
                            ra = rotm + (16 * a)
                            hc = parityd + ra
                            for h in range(2):
                                vals = [
                                    plsc.load_gather(
                                        src, [ra, lconst[8 * h + b]])
                                    for b in range(8)
                                ]
                                for b in range(8):
                                    plsc.store_scatter(
                                        dst, [halfrow[b] + (64 * h), hc],
                                        vals[b])
                        return c2

                    lax.fori_loop(0, L, m_body, 0, unroll=2)

                    @pl.when(q < n_pair)
                    def _():
                        pltpu.async_copy(
                            dst, out_hbm.at[pl.ds(q * 2 * D, 2 * D)],
                            osems[p])

                    @pl.when(q == n_pair)
                    def _():
                        pltpu.async_copy(
                            dst.at[pl.ds(0, D)],
                            out_hbm.at[pl.ds(q * 2 * D, D)], osems[p])

            return carry

        lax.fori_loop(0, t_rounds // 2, round_body, 0)

        for p in range(2):
            qlast = wid + (t_rounds - 2 + p) * NW

            @pl.when(qlast < n_pair)
            def _(p=p, qlast=qlast):
                full_out(qlast, p).wait()

            @pl.when(qlast == n_pair)
            def _(p=p, qlast=qlast):
                pltpu.make_async_copy(
                    dsts[p].at[pl.ds(0, D)],
                    out_hbm.at[pl.ds(qlast * 2 * D, D)], osems[p]
                ).wait()

            @pl.when((qlast > n_pair) & (qlast - 2 * NW <= n_pair)
                     & (qlast >= 2 * NW))
            def _(p=p, qlast=qlast):
                full_out(qlast - 2 * NW, p).wait()

    return relayout


def _make_lookup(V, D, B, H):
    n_blk = B // 128
    bpw = n_blk // NW
    n_chunk = bpw * H
    mesh = plsc.VectorSubcoreMesh(core_axis_name="c", subcore_axis_name="s")

    @functools.partial(
        pl.kernel,
        mesh=mesh,
        out_type=jax.ShapeDtypeStruct((H * 8, n_blk, 1024), jnp.float32),
        scratch_types=[
            pltpu.VMEM((H, 128 * bpw), jnp.int32),
            pltpu.VMEM((128, D), jnp.float32),
            pltpu.VMEM((128, D), jnp.float32),
            pltpu.VMEM((8, 1024), jnp.float32),
            pltpu.VMEM((8, 1024), jnp.float32),
            pltpu.SemaphoreType.DMA,
            pltpu.SemaphoreType.DMA,
            pltpu.SemaphoreType.DMA,
            pltpu.SemaphoreType.DMA,
            pltpu.SemaphoreType.DMA,
        ],
        compiler_params=pltpu.CompilerParams(
            use_tc_tiling_on_sc=False, needs_layout_passes=False
        ),
    )
    def lookup(tlin_hbm, xt_hbm, out_hbm, xball, emb0, emb1, dstv0, dstv1,
               xsem, gsem0, gsem1, osem0, osem1):
        wid = lax.axis_index("s") * NC + lax.axis_index("c")
        embs, dsts = (emb0, emb1), (dstv0, dstv1)
        gsems, osems = (gsem0, gsem1), (osem0, osem1)

        iot = _iota16()
        lconst = [iot + (16 * b) for b in range(8)]

        pltpu.async_copy(
            xt_hbm.at[:, pl.ds(wid * (128 * bpw), 128 * bpw)], xball, xsem
        ).wait()

        def fire_gather(j, bi, p):
            pltpu.async_copy(
                tlin_hbm.at[xball.at[j, pl.ds(bi * 128, 128)]],
                embs[p], gsems[p],
            )

        fire_gather(0, 0, 0)

        def chunk_body(u, carry):
            j, bi = carry
            for p in range(2):
                c = 2 * u + p
                jn = j + 1
                wrap = jn == H
                jn = jnp.where(wrap, 0, jn)
                bn = bi + wrap.astype(jnp.int32)

                @pl.when(c + 1 < n_chunk)
                def _(jn=jn, bn=bn, p=p):
                    fire_gather(jn, bn, 1 - p)

                pltpu.make_async_copy(
                    tlin_hbm.at[xball.at[0, pl.ds(0, 128)]], embs[p], gsems[p]
                ).wait()

                @pl.when(c >= 2)
                def _(p=p):
                    pltpu.make_async_copy(
                        dsts[p], out_hbm.at[pl.ds(0, 8), 0], osems[p]
                    ).wait()

                emb, dst = embs[p], dsts[p]

                def m_body(m, c3):
                    rotm = (iot + m) & 15
                    r8 = lax.shift_right_logical(rotm, 3)
                    rm7 = (rotm & 7) * 128
                    j1b = [rm7 + lconst[b] for b in range(8)]
                    for a in range(D // L):
                        da = rotm + (16 * a)
                        j0a = r8 + (2 * a)
                        vals = [
                            plsc.load_gather(emb, [lconst[b], da])
                            for b in range(8)
                        ]
                        for b in range(8):
                            plsc.store_scatter(dst, [j0a, j1b[b]], vals[b])
                    return c3

                lax.fori_loop(0, L, m_body, 0, unroll=4)

                pltpu.async_copy(
                    dst,
                    out_hbm.at[pl.ds(8 * j, 8), wid * bpw + bi],
                    osems[p],
                )
                j, bi = jn, bn
            return (j, bi)

        lax.fori_loop(0, n_chunk // 2, chunk_body,
                      (jnp.int32(0), jnp.int32(0)))

        for p in range(2):
            pltpu.make_async_copy(
                dsts[p], out_hbm.at[pl.ds(0, 8), 0], osems[p]
            ).wait()

    return lookup


def kernel(x, table):
    B, H = x.shape
    V, D = table.shape
    tt = table.T
    xt = x.astype(jnp.int32).T
    t_lin = _make_relayout(V, D)(tt)
    t64 = t_lin.reshape(t_lin.shape[0] * 2, D)
    out_lin = _make_lookup(V, D, B, H)(t64, xt)
    n_blk = B // 128
    return (out_lin.reshape(H, 8, n_blk, 8, 128)
            .transpose(2, 4, 0, 1, 3)
            .reshape(B, H, D))


# speedup vs baseline: 1.2904x; 1.2904x over previous
import functools

import jax
import jax.numpy as jnp
from jax import lax
from jax.experimental import pallas as pl
from jax.experimental.pallas import tpu as pltpu
from jax.experimental.pallas import tpu_sc as plsc

L = 16
NC = 2
NS = 16
NW = NC * NS


def _iota16():
    return lax.iota(jnp.int32, L)


def _make_relayout(V, D):
    n_vt = (V + 127) // 128
    rows_out = n_vt * D
    rpt = D
    t_rounds = 2 * ((n_vt + NW - 1) // NW + 1) // 2
    mesh = plsc.VectorSubcoreMesh(core_axis_name="c", subcore_axis_name="s")

    @functools.partial(
        pl.kernel,
        mesh=mesh,
        out_type=jax.ShapeDtypeStruct((rows_out, 128), jnp.float32),
        scratch_types=[
            pltpu.VMEM((D, 128), jnp.float32),
            pltpu.VMEM((D, 128), jnp.float32),
            pltpu.VMEM((D, 128), jnp.float32),
            pltpu.VMEM((D, 128), jnp.float32),
            pltpu.SemaphoreType.DMA,
            pltpu.SemaphoreType.DMA,
            pltpu.SemaphoreType.DMA,
            pltpu.SemaphoreType.DMA,
        ],
        compiler_params=pltpu.CompilerParams(
            use_tc_tiling_on_sc=True, needs_layout_passes=False
        ),
    )
    def relayout(tt_hbm, out_hbm, src0, src1, dst0, dst1,
                 isem0, isem1, osem0, osem1):
        wid = lax.axis_index("s") * NC + lax.axis_index("c")
        srcs, dsts = (src0, src1), (dst0, dst1)
        isems, osems = (isem0, isem1), (osem0, osem1)

        iot = _iota16()
        halfrow = [jnp.full((L,), 8 * b, jnp.int32) + (iot // 2)
                   for b in range(8)]
        lconst = [iot + (16 * b) for b in range(8)]
        parityd = (iot & 1) * D

        def fire_in(j, p):
            @pl.when(j < n_vt)
            def _():
                pltpu.async_copy(
                    tt_hbm.at[:, pl.ds(j * 128, 128)], srcs[p], isems[p]
                )

        def out_slice(j):
            return out_hbm.at[pl.ds(j * rpt, rpt)]

        fire_in(wid, 0)

        def round_body(u, carry):
            for p in range(2):
                t = 2 * u + p
                j = wid + t * NW

                fire_in(j + NW, 1 - p)

                @pl.when(j < n_vt)
                def _(t=t, j=j, p=p):
                    pltpu.make_async_copy(
                        tt_hbm.at[:, pl.ds(j * 128, 128)], srcs[p], isems[p]
                    ).wait()

                    @pl.when(t >= 2)
                    def _():
                        pltpu.make_async_copy(
                            dsts[p], out_slice(j - 2 * NW), osems[p]
                        ).wait()

                    src, dst = srcs[p], dsts[p]

                    def m_body(m, c2):
                        rotm = (iot + m) & 15
                        for a in range(D // L):
                            ra = rotm + (16 * a)
                            hc = parityd + ra
                            vals = [
                                plsc.load_gather(src, [ra, lconst[b]])
                                for b in range(8)
                            ]
                            for b in range(8):
                                plsc.store_scatter(
                                    dst, [halfrow[b], hc], vals[b]
                                )
                        return c2

                    lax.fori_loop(0, L, m_body, 0, unroll=4)

                    pltpu.async_copy(dst, out_slice(j), osems[p])

            return carry

        lax.fori_loop(0, t_rounds // 2, round_body, 0)

        for p in range(2):
            jlast_p = wid + (t_rounds - 2 + p) * NW

            @pl.when(jlast_p < n_vt)
            def _(p=p, jlast_p=jlast_p):
                pltpu.make_async_copy(
                    dsts[p], out_slice(jlast_p), osems[p]
                ).wait()

            @pl.when((jlast_p >= n_vt) & (jlast_p - 2 * NW < n_vt)
                     & (jlast_p >= 2 * NW))
            def _(p=p, jlast_p=jlast_p):
                pltpu.make_async_copy(
                    dsts[p], out_slice(jlast_p - 2 * NW), osems[p]
                ).wait()

    return relayout


def _make_lookup(V, D, B, H):
    n_blk = B // 128
    bpw = n_blk // NW
    n_chunk = bpw * H
    mesh = plsc.VectorSubcoreMesh(core_axis_name="c", subcore_axis_name="s")

    @functools.partial(
        pl.kernel,
        mesh=mesh,
        out_type=jax.ShapeDtypeStruct((H * 8, n_blk, 1024), jnp.float32),
        scratch_types=[
            pltpu.VMEM((H, 128 * bpw), jnp.int32),
            pltpu.VMEM((128, D), jnp.float32),
            pltpu.VMEM((128, D), jnp.float32),
            pltpu.VMEM((8, 1024), jnp.float32),
            pltpu.VMEM((8, 1024), jnp.float32),
            pltpu.SemaphoreType.DMA,
            pltpu.SemaphoreType.DMA,
            pltpu.SemaphoreType.DMA,
            pltpu.SemaphoreType.DMA,
            pltpu.SemaphoreType.DMA,
        ],
        compiler_params=pltpu.CompilerParams(
            use_tc_tiling_on_sc=False, needs_layout_passes=False
        ),
    )
    def lookup(tlin_hbm, xt_hbm, out_hbm, xball, emb0, emb1, dstv0, dstv1,
               xsem, gsem0, gsem1, osem0, osem1):
        wid = lax.axis_index("s") * NC + lax.axis_index("c")
        embs, dsts = (emb0, emb1), (dstv0, dstv1)
        gsems, osems = (gsem0, gsem1), (osem0, osem1)

        iot = _iota16()
        lconst = [iot + (16 * b) for b in range(8)]

        pltpu.async_copy(
            xt_hbm.at[:, pl.ds(wid * (128 * bpw), 128 * bpw)], xball, xsem
        ).wait()

        def fire_gather(j, bi, p):
            pltpu.async_copy(
                tlin_hbm.at[xball.at[j, pl.ds(bi * 128, 128)]],
                embs[p], gsems[p],
            )

        fire_gather(0, 0, 0)

        def chunk_body(u, carry):
            j, bi = carry
            for p in range(2):
                c = 2 * u + p
                jn = j + 1
                wrap = jn == H
                jn = jnp.where(wrap, 0, jn)
                bn = bi + wrap.astype(jnp.int32)

                @pl.when(c + 1 < n_chunk)
                def _(jn=jn, bn=bn, p=p):
                    fire_gather(jn, bn, 1 - p)

                pltpu.make_async_copy(
                    tlin_hbm.at[xball.at[0, pl.ds(0, 128)]], embs[p], gsems[p]
                ).wait()

                @pl.when(c >= 2)
                def _(p=p):
                    pltpu.make_async_copy(
                        dsts[p], out_hbm.at[pl.ds(0, 8), 0], osems[p]
                    ).wait()

                emb, dst = embs[p], dsts[p]

                def m_body(m, c3):
                    rotm = (iot + m) & 15
                    r8 = lax.shift_right_logical(rotm, 3)
                    rm7 = (rotm & 7) * 128
                    j1b = [rm7 + lconst[b] for b in range(8)]
                    for a in range(D // L):
                        da = rotm + (16 * a)
                        j0a = r8 + (2 * a)
                        vals = [
                            plsc.load_gather(emb, [lconst[b], da])
                            for b in range(8)
                        ]
                        for b in range(8):
                            plsc.store_scatter(dst, [j0a, j1b[b]], vals[b])
                    return c3

                lax.fori_loop(0, L, m_body, 0, unroll=4)

                pltpu.async_copy(
                    dst,
                    out_hbm.at[pl.ds(8 * j, 8), wid * bpw + bi],
                    osems[p],
                )
                j, bi = jn, bn
            return (j, bi)

        lax.fori_loop(0, n_chunk // 2, chunk_body,
                      (jnp.int32(0), jnp.int32(0)))

        for p in range(2):
            pltpu.make_async_copy(
                dsts[p], out_hbm.at[pl.ds(0, 8), 0], osems[p]
            ).wait()

    return lookup


def kernel(x, table):
    B, H = x.shape
    V, D = table.shape
    tt = table.T
    xt = x.astype(jnp.int32).T
    t_lin = _make_relayout(V, D)(tt)
    t64 = t_lin.reshape(t_lin.shape[0] * 2, D)
    out_lin = _make_lookup(V, D, B, H)(t64, xt)
    n_blk = B // 128
    return (out_lin.reshape(H, 8, n_blk, 8, 128)
            .transpose(2, 4, 0, 1, 3)
            .reshape(B, H, D))
